# baseline (speedup 1.0000x reference)
import functools

import jax
import jax.numpy as jnp
from jax import lax
from jax.experimental import pallas as pl
from jax.experimental.pallas import tpu as pltpu
from jax.experimental.pallas import tpu_sc as plsc

EMB = 64
G = 50
G_PAD = 64
L = 200
CHUNK = 8
N_ITEM_WORKERS = L // CHUNK
NUM_CORES = 2


def _sc_body(gm_hbm, hist_hbm, user_hbm, item_hbm, gout_hbm, iout_hbm,
             idx_v, rows_v, gidx_v, urows_v, gsum_v, sem):
    wid = lax.axis_index("s") * NUM_CORES + lax.axis_index("c")

    @pl.when(wid < N_ITEM_WORKERS)
    def _items():
        base = wid * CHUNK
        pltpu.sync_copy(hist_hbm.at[pl.ds(base, CHUNK)], idx_v)
        pltpu.async_copy(item_hbm.at[idx_v], rows_v, sem).wait()
        pltpu.sync_copy(rows_v, iout_hbm.at[pl.ds(base, CHUNK)])

    @pl.when(wid == N_ITEM_WORKERS)
    def _group():
        pltpu.sync_copy(gm_hbm, gidx_v)
        pltpu.async_copy(user_hbm.at[gidx_v], urows_v, sem).wait()

        def body(i, accs):
            return tuple(a + urows_v[i, pl.ds(j * 16, 16)]
                         for j, a in enumerate(accs))

        accs = tuple(jnp.zeros((16,), jnp.float32) for _ in range(4))
        accs = lax.fori_loop(0, G, body, accs)
        for j in range(4):
            gsum_v[pl.ds(j * 16, 16)] = accs[j]
        pltpu.sync_copy(gsum_v, gout_hbm)


@jax.jit
def _run(gm_pad, history, user_table, item_table):
    mesh = plsc.VectorSubcoreMesh(core_axis_name="c", subcore_axis_name="s")
    k = functools.partial(
        pl.kernel,
        out_type=(jax.ShapeDtypeStruct((EMB,), jnp.float32),
                  jax.ShapeDtypeStruct((L, EMB), jnp.float32)),
        mesh=mesh,
        scratch_types=[
            pltpu.VMEM((CHUNK,), jnp.int32),
            pltpu.VMEM((CHUNK, EMB), jnp.float32),
            pltpu.VMEM((G_PAD,), jnp.int32),
            pltpu.VMEM((G_PAD, EMB), jnp.float32),
            pltpu.VMEM((EMB,), jnp.float32),
            pltpu.SemaphoreType.DMA,
        ],
        compiler_params=pltpu.CompilerParams(use_tc_tiling_on_sc=False),
    )(_sc_body)
    return k(gm_pad, history, user_table, item_table)


def kernel(group_members, history, user_table, item_table, W1, b1, W2, b2):
    gm_pad = jnp.concatenate(
        [group_members, jnp.zeros((G_PAD - G,), group_members.dtype)])
    gout, iout = _run(gm_pad, history, user_table, item_table)
    return jnp.concatenate([gout, iout.reshape(-1)])


# speedup vs baseline: 1.0017x; 1.0017x over previous
import functools

import jax
import jax.numpy as jnp
from jax import lax
from jax.experimental import pallas as pl
from jax.experimental.pallas import tpu as pltpu
from jax.experimental.pallas import tpu_sc as plsc

EMB = 64
G = 50
G_PAD = 56
L = 200
CHUNK = 8
N_ITEM_WORKERS = L // CHUNK
NUM_CORES = 2


def _sc_body(gm_hbm, hist_hbm, user_hbm, item_hbm, gout_hbm, iout_hbm,
             idx_v, rows_v, gidx_v, urows_v, gsum_v, sem):
    wid = lax.axis_index("s") * NUM_CORES + lax.axis_index("c")

    @pl.when(wid < N_ITEM_WORKERS)
    def _items():
        base = wid * CHUNK
        pltpu.sync_copy(hist_hbm.at[pl.ds(base, CHUNK)], idx_v)
        pltpu.async_copy(item_hbm.at[idx_v], rows_v, sem).wait()
        pltpu.sync_copy(rows_v, iout_hbm.at[pl.ds(base, CHUNK)])

    @pl.when(wid == N_ITEM_WORKERS)
    def _group():
        pltpu.sync_copy(gm_hbm, gidx_v)
        pltpu.async_copy(user_hbm.at[gidx_v], urows_v, sem).wait()
        accs = [jnp.zeros((16,), jnp.float32) for _ in range(4)]
        for j in range(G):
            for k in range(4):
                accs[k] = accs[k] + urows_v[j, pl.ds(k * 16, 16)]
        for k in range(4):
            gsum_v[pl.ds(k * 16, 16)] = accs[k]
        pltpu.sync_copy(gsum_v, gout_hbm)


@jax.jit
def _run(gm_pad, history, user_table, item_table):
    mesh = plsc.VectorSubcoreMesh(core_axis_name="c", subcore_axis_name="s")
    k = functools.partial(
        pl.kernel,
        out_type=(jax.ShapeDtypeStruct((EMB,), jnp.float32),
                  jax.ShapeDtypeStruct((L, EMB), jnp.float32)),
        mesh=mesh,
        compiler_params=pltpu.CompilerParams(use_tc_tiling_on_sc=False),
        scratch_types=[
            pltpu.VMEM((CHUNK,), jnp.int32),
            pltpu.VMEM((CHUNK, EMB), jnp.float32),
            pltpu.VMEM((G_PAD,), jnp.int32),
            pltpu.VMEM((G_PAD, EMB), jnp.float32),
            pltpu.VMEM((EMB,), jnp.float32),
            pltpu.SemaphoreType.DMA,
        ],
    )(_sc_body)
    return k(gm_pad, history, user_table, item_table)


def kernel(group_members, history, user_table, item_table, W1, b1, W2, b2):
    gm_pad = jnp.concatenate(
        [group_members, jnp.zeros((G_PAD - G,), group_members.dtype)])
    gout, iout = _run(gm_pad, history, user_table, item_table)
    return jnp.concatenate([gout, iout.reshape(-1)])


# speedup vs baseline: 1.5561x; 1.5535x over previous
import functools

import jax
import jax.numpy as jnp
from jax import lax
from jax.experimental import pallas as pl
from jax.experimental.pallas import tpu as pltpu
from jax.experimental.pallas import tpu_sc as plsc

EMB = 64
G = 50
G_PAD = 56
L = 200
NUM_CORES = 2


def _sc_body(gm_hbm, user_hbm, gout_hbm, gidx_v, urows_v, gsum_v, sem):
    wid = lax.axis_index("s") * NUM_CORES + lax.axis_index("c")

    @pl.when(wid == 0)
    def _group():
        pltpu.sync_copy(gm_hbm, gidx_v)
        pltpu.async_copy(user_hbm.at[gidx_v], urows_v, sem).wait()
        accs = [jnp.zeros((16,), jnp.float32) for _ in range(4)]
        for j in range(G):
            for k in range(4):
                accs[k] = accs[k] + urows_v[j, pl.ds(k * 16, 16)]
        for k in range(4):
            gsum_v[pl.ds(k * 16, 16)] = accs[k]
        pltpu.sync_copy(gsum_v, gout_hbm)


def _tc_body(hist_smem, item_hbm, out_vmem, sem):
    copies = []
    for j in range(L):
        i = hist_smem[j]
        c = pltpu.make_async_copy(item_hbm.at[pl.ds(i, 1)],
                                  out_vmem.at[pl.ds(j, 1)], sem)
        c.start()
        copies.append(c)
    for c in copies:
        c.wait()


@jax.jit
def _run(gm_pad, history, user_table, item_table):
    mesh = plsc.VectorSubcoreMesh(core_axis_name="c", subcore_axis_name="s")
    sc_k = functools.partial(
        pl.kernel,
        out_type=jax.ShapeDtypeStruct((EMB,), jnp.float32),
        mesh=mesh,
        compiler_params=pltpu.CompilerParams(use_tc_tiling_on_sc=False),
        scratch_types=[
            pltpu.VMEM((G_PAD,), jnp.int32),
            pltpu.VMEM((G_PAD, EMB), jnp.float32),
            pltpu.VMEM((EMB,), jnp.float32),
            pltpu.SemaphoreType.DMA,
        ],
    )(_sc_body)
    gout = sc_k(gm_pad, user_table)

    iout = pl.pallas_call(
        _tc_body,
        grid_spec=pltpu.PrefetchScalarGridSpec(
            num_scalar_prefetch=1,
            grid=(1,),
            in_specs=[pl.BlockSpec(memory_space=pl.ANY)],
            out_specs=pl.BlockSpec(memory_space=pltpu.VMEM),
            scratch_shapes=[pltpu.SemaphoreType.DMA],
        ),
        out_shape=jax.ShapeDtypeStruct((L, EMB), jnp.float32),
    )(history, item_table)
    return gout, iout


def kernel(group_members, history, user_table, item_table, W1, b1, W2, b2):
    gm_pad = jnp.concatenate(
        [group_members, jnp.zeros((G_PAD - G,), group_members.dtype)])
    gout, iout = _run(gm_pad, history, user_table, item_table)
    return jnp.concatenate([gout, iout.reshape(-1)])
